# baseline (speedup 1.0000x reference)
import functools

import jax
import jax.numpy as jnp
from jax import lax
from jax.experimental import pallas as pl
from jax.experimental.pallas import tpu as pltpu
from jax.experimental.pallas import tpu_sc as plsc

_HID = 128
_NC = 2
_NS = 16
_NW = _NC * _NS
_LANES = 16
_CHUNK = 200
_GRP = 25


def _edge_sc(xe, tab_flat):
    E = xe.shape[0]
    rows_w = E // _NW
    n_ch = rows_w // _CHUNK
    ncol = _HID // _LANES

    mesh = plsc.VectorSubcoreMesh(core_axis_name="c", subcore_axis_name="s")

    @functools.partial(
        pl.kernel,
        mesh=mesh,
        compiler_params=pltpu.CompilerParams(needs_layout_passes=False),
        out_type=jax.ShapeDtypeStruct((E * _HID,), jnp.float32),
        scratch_types=(
            [pltpu.VMEM((_CHUNK,), jnp.int32) for _ in range(4)]
            + [pltpu.VMEM((_CHUNK * _HID,), jnp.float32) for _ in range(4)]
            + [pltpu.VMEM((2 * _HID,), jnp.float32)]
            + [pltpu.VMEM_SHARED((_NS, _CHUNK * _HID), jnp.float32)]
            + [pltpu.SemaphoreType.DMA for _ in range(8)]
        ),
    )
    def k(xe_hbm, tab_hbm, out_hbm, *scratch):
        idxb = scratch[0:4]
        outd = scratch[4:6]
        sbuf = scratch[6:8]
        tab_v = scratch[8]
        spm = scratch[9]
        isem = scratch[10:14]
        osem = scratch[14:16]
        s2sem = scratch[16:18]
        cid = lax.axis_index("c")
        sid = lax.axis_index("s")
        wid = sid * _NC + cid
        base = wid * rows_w
        pltpu.sync_copy(tab_hbm, tab_v)
        r0 = [
            jnp.maximum(tab_v[pl.ds(_LANES * j, _LANES)], 0.0)
            for j in range(ncol)
        ]
        r1 = [
            jnp.maximum(tab_v[pl.ds(_HID + _LANES * j, _LANES)], 0.0)
            for j in range(ncol)
        ]

        for b in range(4):
            pltpu.async_copy(
                xe_hbm.at[pl.ds(base + b * _CHUNK, _CHUNK)], idxb[b], isem[b]
            )

        def compute(ib, ob):
            def grp(g, c2):
                for i in range(_GRP):
                    r = g * _GRP + i
                    sp = plsc.load_gather(
                        ib, [jnp.broadcast_to(r, (_LANES,))]
                    )
                    m = sp == 0
                    for j in range(ncol):
                        ob[pl.ds(r * _HID + _LANES * j, _LANES)] = (
                            jnp.where(m, r0[j], r1[j])
                        )
                return c2

            lax.fori_loop(0, _CHUNK // _GRP, grp, 0, unroll=False)

        def wait_idx(ch, j):
            pltpu.make_async_copy(
                xe_hbm.at[pl.ds(base + ch * _CHUNK, _CHUNK)],
                idxb[j], isem[j],
            ).wait()

        def next_idx(ch, j):
            @pl.when(ch + 4 < n_ch)
            def _():
                pltpu.async_copy(
                    xe_hbm.at[pl.ds(base + (ch + 4) * _CHUNK, _CHUNK)],
                    idxb[j], isem[j],
                )

        def direct_chunk(ch, k2, j):
            row0 = base + ch * _CHUNK
            wait_idx(ch, j)

            @pl.when(ch >= 4)
            def _wait_store():
                pltpu.make_async_copy(
                    outd[k2],
                    out_hbm.at[pl.ds(row0 * _HID, _CHUNK * _HID)],
                    osem[k2],
                ).wait()

            compute(idxb[j], outd[k2])
            pltpu.async_copy(
                outd[k2],
                out_hbm.at[pl.ds(row0 * _HID, _CHUNK * _HID)],
                osem[k2],
            )
            next_idx(ch, j)

        def spmem_chunk(ch, k2, j):
            row0 = base + ch * _CHUNK
            wait_idx(ch, j)
            compute(idxb[j], sbuf[k2])

            @pl.when(ch >= 2)
            def _wait_hbm():
                pltpu.make_async_copy(
                    spm.at[sid],
                    out_hbm.at[pl.ds(row0 * _HID, _CHUNK * _HID)],
                    s2sem[0],
                ).wait()

            pltpu.sync_copy(sbuf[k2], spm.at[sid])
            pltpu.async_copy(
                spm.at[sid],
                out_hbm.at[pl.ds(row0 * _HID, _CHUNK * _HID)],
                s2sem[0],
            )
            next_idx(ch, j)

        def quad_body(t, carry):
            ch0 = t * 4
            direct_chunk(ch0, 0, 0)
            spmem_chunk(ch0 + 1, 0, 1)
            direct_chunk(ch0 + 2, 1, 2)
            spmem_chunk(ch0 + 3, 1, 3)
            return carry

        lax.fori_loop(0, n_ch // 4, quad_body, 0, unroll=False)
        rem_base = (n_ch // 4) * 4
        for ch in range(rem_base, n_ch):
            if ch % 2 == 0:
                direct_chunk(ch, (ch // 2) % 2, ch % 4)
            else:
                spmem_chunk(ch, (ch // 2) % 2, ch % 4)

        d_tail = {0: None, 1: None}
        s_tail = {0: None, 1: None}
        for ch in range(n_ch):
            if ch % 2 == 0:
                d_tail[(ch // 2) % 2] = ch
            else:
                s_tail[(ch // 2) % 2] = ch
        for k2 in range(2):
            if d_tail[k2] is not None:
                last0 = base + d_tail[k2] * _CHUNK
                pltpu.make_async_copy(
                    outd[k2],
                    out_hbm.at[pl.ds(last0 * _HID, _CHUNK * _HID)],
                    osem[k2],
                ).wait()
            if k2 == 0 and s_tail[0] is not None or k2 == 1 and False:
                last0 = base + max(v for v in s_tail.values() if v is not None) * _CHUNK
                pltpu.make_async_copy(
                    spm.at[sid],
                    out_hbm.at[pl.ds(last0 * _HID, _CHUNK * _HID)],
                    s2sem[0],
                ).wait()

    return k(xe, tab_flat)


def _node_tc(xg, xc, ecat_pad, wg, bg, w1, w2, bn):
    N = xg.shape[0]
    BN = 1000
    CPAD = ecat_pad.shape[0]

    def body(xg_ref, xc_ref, ec_ref, wg_ref, bg_ref, w1_ref, w2_ref,
             bn_ref, out_ref):
        g = jnp.maximum(
            jnp.dot(xg_ref[...], wg_ref[...],
                    preferred_element_type=jnp.float32) + bg_ref[...],
            0.0,
        )
        ids = xc_ref[...]
        oh = (ids == lax.broadcasted_iota(jnp.int32, (BN, CPAD), 1)
              ).astype(jnp.float32)
        cat = jnp.maximum(
            jnp.dot(oh, ec_ref[...], preferred_element_type=jnp.float32),
            0.0,
        )
        out = (
            jnp.dot(g, w1_ref[...], preferred_element_type=jnp.float32)
            + jnp.dot(cat, w2_ref[...], preferred_element_type=jnp.float32)
            + bn_ref[...]
        )
        out_ref[...] = jnp.maximum(out, 0.0)

    return pl.pallas_call(
        body,
        grid=(N // BN,),
        in_specs=[
            pl.BlockSpec((BN, 16), lambda i: (i, 0)),
            pl.BlockSpec((BN, 1), lambda i: (i, 0)),
            pl.BlockSpec((CPAD, _HID), lambda i: (0, 0)),
            pl.BlockSpec((16, _HID), lambda i: (0, 0)),
            pl.BlockSpec((1, _HID), lambda i: (0, 0)),
            pl.BlockSpec((_HID, _HID), lambda i: (0, 0)),
            pl.BlockSpec((_HID, _HID), lambda i: (0, 0)),
            pl.BlockSpec((1, _HID), lambda i: (0, 0)),
        ],
        out_specs=pl.BlockSpec((BN, _HID), lambda i: (i, 0)),
        out_shape=jax.ShapeDtypeStruct((N, _HID), jnp.float32),
    )(xg, xc, ecat_pad, wg, bg, w1, w2, bn)


@jax.jit
def kernel(xn_geom, xn_cat, xe, E_cat, W_geom, b_geom, W_node, b_node,
           E_edge):
    E = xe.shape[0]
    cats = E_cat.shape[0]
    cpad = ((cats + 127) // 128) * 128

    xe_i32 = xe.astype(jnp.int32)
    tab_flat = E_edge.reshape(-1)
    xe_flat = _edge_sc(xe_i32, tab_flat)
    xe_out = xe_flat.reshape(E, _HID)

    ecat_pad = jnp.concatenate(
        [E_cat, jnp.zeros((cpad - cats, _HID), jnp.float32)], axis=0
    )
    w1 = W_node[:_HID]
    w2 = W_node[_HID:]
    xn = _node_tc(
        xn_geom,
        xn_cat.astype(jnp.int32),
        ecat_pad,
        W_geom,
        b_geom.reshape(1, _HID),
        w1,
        w2,
        b_node.reshape(1, _HID),
    )
    return (xn, xe_out)


# speedup vs baseline: 1.2345x; 1.2345x over previous
import functools

import jax
import jax.numpy as jnp
from jax import lax
from jax.experimental import pallas as pl
from jax.experimental.pallas import tpu as pltpu
from jax.experimental.pallas import tpu_sc as plsc

_HID = 128
_NC = 2
_NS = 16
_NW = _NC * _NS
_LANES = 16
_CHUNK = 200
_GRP = 25


def _edge_sc(xe, tab_flat):
    E = xe.shape[0]
    rows_w = E // _NW
    n_ch = rows_w // _CHUNK
    ncol = _HID // _LANES

    mesh = plsc.VectorSubcoreMesh(core_axis_name="c", subcore_axis_name="s")

    @functools.partial(
        pl.kernel,
        mesh=mesh,
        compiler_params=pltpu.CompilerParams(needs_layout_passes=False),
        out_type=jax.ShapeDtypeStruct((E * _HID,), jnp.float32),
        scratch_types=[
            pltpu.VMEM((_CHUNK,), jnp.int32),
            pltpu.VMEM((_CHUNK,), jnp.int32),
            pltpu.VMEM((_CHUNK * _HID,), jnp.float32),
            pltpu.VMEM((_CHUNK * _HID,), jnp.float32),
            pltpu.VMEM((2 * _HID,), jnp.float32),
            pltpu.SemaphoreType.DMA,
            pltpu.SemaphoreType.DMA,
            pltpu.SemaphoreType.DMA,
            pltpu.SemaphoreType.DMA,
        ],
    )
    def k(xe_hbm, tab_hbm, out_hbm, idx0, idx1, outv0, outv1, tab_v,
          is0, is1, os0, os1):
        idxb = (idx0, idx1)
        outb = (outv0, outv1)
        isem = (is0, is1)
        osem = (os0, os1)
        wid = lax.axis_index("s") * _NC + lax.axis_index("c")
        base = wid * rows_w
        pltpu.sync_copy(tab_hbm, tab_v)
        r0 = [
            jnp.maximum(tab_v[pl.ds(_LANES * j, _LANES)], 0.0)
            for j in range(ncol)
        ]
        r1 = [
            jnp.maximum(tab_v[pl.ds(_HID + _LANES * j, _LANES)], 0.0)
            for j in range(ncol)
        ]

        for b in range(2):
            pltpu.async_copy(
                xe_hbm.at[pl.ds(base + b * _CHUNK, _CHUNK)], idxb[b], isem[b]
            )

        def pair_body(t, carry):
            ch0 = t * 2
            for b in range(2):
                ch = ch0 + b
                row0 = base + ch * _CHUNK
                pltpu.make_async_copy(
                    xe_hbm.at[pl.ds(row0, _CHUNK)], idxb[b], isem[b]
                ).wait()

                @pl.when(ch >= 2)
                def _wait_store():
                    pltpu.make_async_copy(
                        outb[b],
                        out_hbm.at[pl.ds(row0 * _HID, _CHUNK * _HID)],
                        osem[b],
                    ).wait()

                def grp(g, c2):
                    for i in range(_GRP):
                        r = g * _GRP + i
                        sp = plsc.load_gather(
                            idxb[b], [jnp.broadcast_to(r, (_LANES,))]
                        )
                        m = sp == 0
                        for j in range(ncol):
                            outb[b][pl.ds(r * _HID + _LANES * j, _LANES)] = (
                                jnp.where(m, r0[j], r1[j])
                            )
                    return c2

                lax.fori_loop(0, _CHUNK // _GRP, grp, 0, unroll=False)
                pltpu.async_copy(
                    outb[b],
                    out_hbm.at[pl.ds(row0 * _HID, _CHUNK * _HID)],
                    osem[b],
                )

                @pl.when(ch + 2 < n_ch)
                def _next_idx():
                    pltpu.async_copy(
                        xe_hbm.at[pl.ds(row0 + 2 * _CHUNK, _CHUNK)],
                        idxb[b],
                        isem[b],
                    )

            return carry

        lax.fori_loop(0, n_ch // 2, pair_body, 0, unroll=False)
        for b in range(2):
            last0 = base + (n_ch - 2 + b) * _CHUNK
            pltpu.make_async_copy(
                outb[b],
                out_hbm.at[pl.ds(last0 * _HID, _CHUNK * _HID)],
                osem[b],
            ).wait()

    return k(xe, tab_flat)


def _node_tc(xg, xc, ecat_pad, wg, bg, w1, w2, bn):
    N = xg.shape[0]
    BN = 1000
    CPAD = ecat_pad.shape[0]

    def body(xg_ref, xc_ref, ec_ref, wg_ref, bg_ref, w1_ref, w2_ref,
             bn_ref, out_ref):
        g = jnp.maximum(
            jnp.dot(xg_ref[...], wg_ref[...],
                    preferred_element_type=jnp.float32) + bg_ref[...],
            0.0,
        )
        ids = xc_ref[...]
        oh = (ids == lax.broadcasted_iota(jnp.int32, (BN, CPAD), 1)
              ).astype(jnp.float32)
        cat = jnp.maximum(
            jnp.dot(oh, ec_ref[...], preferred_element_type=jnp.float32),
            0.0,
        )
        out = (
            jnp.dot(g, w1_ref[...], preferred_element_type=jnp.float32)
            + jnp.dot(cat, w2_ref[...], preferred_element_type=jnp.float32)
            + bn_ref[...]
        )
        out_ref[...] = jnp.maximum(out, 0.0)

    return pl.pallas_call(
        body,
        grid=(N // BN,),
        in_specs=[
            pl.BlockSpec((BN, 16), lambda i: (i, 0)),
            pl.BlockSpec((BN, 1), lambda i: (i, 0)),
            pl.BlockSpec((CPAD, _HID), lambda i: (0, 0)),
            pl.BlockSpec((16, _HID), lambda i: (0, 0)),
            pl.BlockSpec((1, _HID), lambda i: (0, 0)),
            pl.BlockSpec((_HID, _HID), lambda i: (0, 0)),
            pl.BlockSpec((_HID, _HID), lambda i: (0, 0)),
            pl.BlockSpec((1, _HID), lambda i: (0, 0)),
        ],
        out_specs=pl.BlockSpec((BN, _HID), lambda i: (i, 0)),
        out_shape=jax.ShapeDtypeStruct((N, _HID), jnp.float32),
    )(xg, xc, ecat_pad, wg, bg, w1, w2, bn)


@jax.jit
def kernel(xn_geom, xn_cat, xe, E_cat, W_geom, b_geom, W_node, b_node,
           E_edge):
    E = xe.shape[0]
    cats = E_cat.shape[0]
    cpad = ((cats + 127) // 128) * 128

    xe_i32 = xe.astype(jnp.int32)
    tab_flat = E_edge.reshape(-1)
    xe_flat = _edge_sc(xe_i32, tab_flat)
    xe_out = xe_flat.reshape(E, _HID)

    ecat_pad = jnp.concatenate(
        [E_cat, jnp.zeros((cpad - cats, _HID), jnp.float32)], axis=0
    )
    w1 = W_node[:_HID]
    w2 = W_node[_HID:]
    xn = _node_tc(
        xn_geom,
        xn_cat.astype(jnp.int32),
        ecat_pad,
        W_geom,
        b_geom.reshape(1, _HID),
        w1,
        w2,
        b_node.reshape(1, _HID),
    )
    return (xn, xe_out)


# speedup vs baseline: 1.2360x; 1.0012x over previous
import functools

import jax
import jax.numpy as jnp
from jax import lax
from jax.experimental import pallas as pl
from jax.experimental.pallas import tpu as pltpu
from jax.experimental.pallas import tpu_sc as plsc

_HID = 128
_NC = 2
_NS = 16
_NW = _NC * _NS
_LANES = 16
_CHUNK = 200
_GRP = 25


def _edge_sc(xe, tab_flat):
    E = xe.shape[0]
    rows_w = E // _NW
    n_ch = rows_w // _CHUNK
    ncol = _HID // _LANES

    mesh = plsc.VectorSubcoreMesh(core_axis_name="c", subcore_axis_name="s")

    @functools.partial(
        pl.kernel,
        mesh=mesh,
        compiler_params=pltpu.CompilerParams(needs_layout_passes=False),
        out_type=jax.ShapeDtypeStruct((E * _HID,), jnp.float32),
        scratch_types=[
            pltpu.VMEM((rows_w,), jnp.int32),
            pltpu.VMEM((_CHUNK * _HID,), jnp.float32),
            pltpu.VMEM((_CHUNK * _HID,), jnp.float32),
            pltpu.VMEM((2 * _HID,), jnp.float32),
            pltpu.SemaphoreType.DMA,
            pltpu.SemaphoreType.DMA,
        ],
    )
    def k(xe_hbm, tab_hbm, out_hbm, idx_v, outv0, outv1, tab_v, os0, os1):
        outb = (outv0, outv1)
        osem = (os0, os1)
        wid = lax.axis_index("s") * _NC + lax.axis_index("c")
        base = wid * rows_w
        pltpu.sync_copy(xe_hbm.at[pl.ds(base, rows_w)], idx_v)
        pltpu.sync_copy(tab_hbm, tab_v)
        r0 = [
            jnp.maximum(tab_v[pl.ds(_LANES * j, _LANES)], 0.0)
            for j in range(ncol)
        ]
        r1 = [
            jnp.maximum(tab_v[pl.ds(_HID + _LANES * j, _LANES)], 0.0)
            for j in range(ncol)
        ]

        def pair_body(t, carry):
            ch0 = t * 2
            for b in range(2):
                ch = ch0 + b
                row0 = base + ch * _CHUNK

                @pl.when(ch >= 2)
                def _wait_store():
                    pltpu.make_async_copy(
                        outb[b],
                        out_hbm.at[pl.ds(row0 * _HID, _CHUNK * _HID)],
                        osem[b],
                    ).wait()

                def grp(g, c2):
                    for i in range(_GRP):
                        r = g * _GRP + i
                        sp = plsc.load_gather(
                            idx_v,
                            [jnp.broadcast_to(
                                ch * _CHUNK + r, (_LANES,))],
                        )
                        m = sp == 0
                        for j in range(ncol):
                            outb[b][pl.ds(r * _HID + _LANES * j, _LANES)] = (
                                jnp.where(m, r0[j], r1[j])
                            )
                    return c2

                lax.fori_loop(0, _CHUNK // _GRP, grp, 0, unroll=False)
                pltpu.async_copy(
                    outb[b],
                    out_hbm.at[pl.ds(row0 * _HID, _CHUNK * _HID)],
                    osem[b],
                )

            return carry

        lax.fori_loop(0, n_ch // 2, pair_body, 0, unroll=False)
        for b in range(2):
            last0 = base + (n_ch - 2 + b) * _CHUNK
            pltpu.make_async_copy(
                outb[b],
                out_hbm.at[pl.ds(last0 * _HID, _CHUNK * _HID)],
                osem[b],
            ).wait()

    return k(xe, tab_flat)


def _node_tc(xg, xc, ecat_pad, wg, bg, w1, w2, bn):
    N = xg.shape[0]
    BN = 1000
    CPAD = ecat_pad.shape[0]

    def body(xg_ref, xc_ref, ec_ref, wg_ref, bg_ref, w1_ref, w2_ref,
             bn_ref, out_ref):
        g = jnp.maximum(
            jnp.dot(xg_ref[...], wg_ref[...],
                    preferred_element_type=jnp.float32) + bg_ref[...],
            0.0,
        )
        ids = xc_ref[...]
        oh = (ids == lax.broadcasted_iota(jnp.int32, (BN, CPAD), 1)
              ).astype(jnp.float32)
        cat = jnp.maximum(
            jnp.dot(oh, ec_ref[...], preferred_element_type=jnp.float32),
            0.0,
        )
        out = (
            jnp.dot(g, w1_ref[...], preferred_element_type=jnp.float32)
            + jnp.dot(cat, w2_ref[...], preferred_element_type=jnp.float32)
            + bn_ref[...]
        )
        out_ref[...] = jnp.maximum(out, 0.0)

    return pl.pallas_call(
        body,
        grid=(N // BN,),
        in_specs=[
            pl.BlockSpec((BN, 16), lambda i: (i, 0)),
            pl.BlockSpec((BN, 1), lambda i: (i, 0)),
            pl.BlockSpec((CPAD, _HID), lambda i: (0, 0)),
            pl.BlockSpec((16, _HID), lambda i: (0, 0)),
            pl.BlockSpec((1, _HID), lambda i: (0, 0)),
            pl.BlockSpec((_HID, _HID), lambda i: (0, 0)),
            pl.BlockSpec((_HID, _HID), lambda i: (0, 0)),
            pl.BlockSpec((1, _HID), lambda i: (0, 0)),
        ],
        out_specs=pl.BlockSpec((BN, _HID), lambda i: (i, 0)),
        out_shape=jax.ShapeDtypeStruct((N, _HID), jnp.float32),
    )(xg, xc, ecat_pad, wg, bg, w1, w2, bn)


@jax.jit
def kernel(xn_geom, xn_cat, xe, E_cat, W_geom, b_geom, W_node, b_node,
           E_edge):
    E = xe.shape[0]
    cats = E_cat.shape[0]
    cpad = ((cats + 127) // 128) * 128

    xe_i32 = xe.astype(jnp.int32)
    tab_flat = E_edge.reshape(-1)
    xe_flat = _edge_sc(xe_i32, tab_flat)
    xe_out = xe_flat.reshape(E, _HID)

    ecat_pad = jnp.concatenate(
        [E_cat, jnp.zeros((cpad - cats, _HID), jnp.float32)], axis=0
    )
    w1 = W_node[:_HID]
    w2 = W_node[_HID:]
    xn = _node_tc(
        xn_geom,
        xn_cat.astype(jnp.int32),
        ecat_pad,
        W_geom,
        b_geom.reshape(1, _HID),
        w1,
        w2,
        b_node.reshape(1, _HID),
    )
    return (xn, xe_out)
